# baseline (device time: 43034 ns/iter reference)
import jax
import jax.numpy as jnp
from jax import lax
from jax.experimental import pallas as pl
from jax.experimental.pallas import tpu as pltpu

N_DEV = 4


def kernel(x, w_mat):
    m_global, k_my = x.shape
    k_global, n = w_mat.shape
    m_per = m_global // N_DEV

    def body(x_ref, w_ref, out_ref, xg_ref, send_sems, recv_sems):
        my = lax.axis_index("i")

        barrier_sem = pltpu.get_barrier_semaphore()
        for off in range(1, N_DEV):
            peer = (my + off) % N_DEV
            pl.semaphore_signal(
                barrier_sem, inc=1,
                device_id=(peer,), device_id_type=pl.DeviceIdType.MESH,
            )
        pl.semaphore_wait(barrier_sem, N_DEV - 1)

        rdmas = []
        for off in range(1, N_DEV):
            peer = (my + off) % N_DEV
            rdma = pltpu.make_async_remote_copy(
                src_ref=x_ref.at[pl.ds(peer * m_per, m_per), :],
                dst_ref=xg_ref.at[:, pl.ds(my * k_my, k_my)],
                send_sem=send_sems.at[off - 1],
                recv_sem=recv_sems.at[off - 1],
                device_id=(peer,),
                device_id_type=pl.DeviceIdType.MESH,
            )
            rdma.start()
            rdmas.append(rdma)

        xg_ref[:, pl.ds(my * k_my, k_my)] = x_ref[pl.ds(my * m_per, m_per), :]

        for rdma in rdmas:
            rdma.wait()

        acc = jnp.dot(xg_ref[:, :], w_ref[:, :],
                      preferred_element_type=jnp.float32)
        c = 0.7978845608028654
        out_ref[:, :] = 0.5 * acc * (1.0 + jnp.tanh(c * (acc + 0.044715 * acc * acc * acc)))

    return pl.pallas_call(
        body,
        out_shape=jax.ShapeDtypeStruct((m_per, n), jnp.float32),
        in_specs=[
            pl.BlockSpec(memory_space=pltpu.VMEM),
            pl.BlockSpec(memory_space=pltpu.VMEM),
        ],
        out_specs=pl.BlockSpec(memory_space=pltpu.VMEM),
        scratch_shapes=[
            pltpu.VMEM((m_per, k_global), jnp.float32),
            pltpu.SemaphoreType.DMA((N_DEV - 1,)),
            pltpu.SemaphoreType.DMA((N_DEV - 1,)),
        ],
        compiler_params=pltpu.CompilerParams(collective_id=0),
    )(x, w_mat)


# device time: 34097 ns/iter; 1.2621x vs baseline; 1.2621x over previous
import jax
import jax.numpy as jnp
from jax import lax
from jax.experimental import pallas as pl
from jax.experimental.pallas import tpu as pltpu

N_DEV = 4


def kernel(x, w_mat):
    m_global, k_my = x.shape
    k_global, n = w_mat.shape
    m_per = m_global // N_DEV

    def body(x_ref, w_ref, out_ref, xg_ref, wv_ref, send_sems, recv_sems, wcopy_sems):
        my = lax.axis_index("i")

        order = [my] + [(my - off) % N_DEV for off in range(1, N_DEV)]

        wcopies = []
        for idx, j in enumerate(order):
            cp = pltpu.make_async_copy(
                w_ref.at[pl.ds(j * k_my, k_my), :],
                wv_ref.at[pl.ds(j * k_my, k_my), :],
                wcopy_sems.at[idx],
            )
            cp.start()
            wcopies.append(cp)

        barrier_sem = pltpu.get_barrier_semaphore()
        for off in range(1, N_DEV):
            peer = (my + off) % N_DEV
            pl.semaphore_signal(
                barrier_sem, inc=1,
                device_id=(peer,), device_id_type=pl.DeviceIdType.MESH,
            )
        pl.semaphore_wait(barrier_sem, N_DEV - 1)

        rdmas = []
        for off in range(1, N_DEV):
            peer = (my + off) % N_DEV
            rdma = pltpu.make_async_remote_copy(
                src_ref=x_ref.at[pl.ds(peer * m_per, m_per), :],
                dst_ref=xg_ref.at[:, pl.ds(my * k_my, k_my)],
                send_sem=send_sems.at[off - 1],
                recv_sem=recv_sems.at[off - 1],
                device_id=(peer,),
                device_id_type=pl.DeviceIdType.MESH,
            )
            rdma.start()
            rdmas.append(rdma)

        wcopies[0].wait()
        out_ref[:, :] = jnp.dot(
            x_ref[pl.ds(my * m_per, m_per), :],
            wv_ref[pl.ds(my * k_my, k_my), :],
            preferred_element_type=jnp.float32,
        )

        for off in range(1, N_DEV):
            rdmas[off - 1].wait()
            wcopies[off].wait()
            j = order[off]
            out_ref[:, :] = out_ref[:, :] + jnp.dot(
                xg_ref[:, pl.ds(j * k_my, k_my)],
                wv_ref[pl.ds(j * k_my, k_my), :],
                preferred_element_type=jnp.float32,
            )

        acc = out_ref[:, :]
        c = 0.7978845608028654
        out_ref[:, :] = 0.5 * acc * (1.0 + jnp.tanh(c * (acc + 0.044715 * acc * acc * acc)))

    return pl.pallas_call(
        body,
        out_shape=jax.ShapeDtypeStruct((m_per, n), jnp.float32),
        in_specs=[
            pl.BlockSpec(memory_space=pltpu.VMEM),
            pl.BlockSpec(memory_space=pl.ANY),
        ],
        out_specs=pl.BlockSpec(memory_space=pltpu.VMEM),
        scratch_shapes=[
            pltpu.VMEM((m_per, k_global), jnp.float32),
            pltpu.VMEM((k_global, n), jnp.float32),
            pltpu.SemaphoreType.DMA((N_DEV - 1,)),
            pltpu.SemaphoreType.DMA((N_DEV - 1,)),
            pltpu.SemaphoreType.DMA((N_DEV,)),
        ],
        compiler_params=pltpu.CompilerParams(collective_id=0),
    )(x, w_mat)


# device time: 22822 ns/iter; 1.8856x vs baseline; 1.4940x over previous
import jax
import jax.numpy as jnp
from jax import lax
from jax.experimental import pallas as pl
from jax.experimental.pallas import tpu as pltpu

N_DEV = 4


def kernel(x, w_mat):
    m_global, k_my = x.shape
    k_global, n = w_mat.shape
    m_per = m_global // N_DEV

    def body(x_ref, w_ref, out_ref, xb_ref, xg_ref, wv_ref, wb_ref,
             send_sems, recv_sems, wcopy_sems):
        my = lax.axis_index("i")

        order = [my] + [(my - off) % N_DEV for off in range(1, N_DEV)]

        wcopies = []
        for idx, j in enumerate(order):
            cp = pltpu.make_async_copy(
                w_ref.at[pl.ds(j * k_my, k_my), :],
                wv_ref.at[pl.ds(j * k_my, k_my), :],
                wcopy_sems.at[idx],
            )
            cp.start()
            wcopies.append(cp)

        barrier_sem = pltpu.get_barrier_semaphore()
        for off in range(1, N_DEV):
            peer = (my + off) % N_DEV
            pl.semaphore_signal(
                barrier_sem, inc=1,
                device_id=(peer,), device_id_type=pl.DeviceIdType.MESH,
            )

        xb_ref[:, :] = x_ref[:, :].astype(jnp.bfloat16)

        pl.semaphore_wait(barrier_sem, N_DEV - 1)

        rdmas = []
        for off in range(1, N_DEV):
            peer = (my + off) % N_DEV
            rdma = pltpu.make_async_remote_copy(
                src_ref=xb_ref.at[pl.ds(peer * m_per, m_per), :],
                dst_ref=xg_ref.at[:, pl.ds(my * k_my, k_my)],
                send_sem=send_sems.at[off - 1],
                recv_sem=recv_sems.at[off - 1],
                device_id=(peer,),
                device_id_type=pl.DeviceIdType.MESH,
            )
            rdma.start()
            rdmas.append(rdma)

        wcopies[0].wait()
        wb_ref[pl.ds(my * k_my, k_my), :] = (
            wv_ref[pl.ds(my * k_my, k_my), :].astype(jnp.bfloat16))
        out_ref[:, :] = jnp.dot(
            xb_ref[pl.ds(my * m_per, m_per), :],
            wb_ref[pl.ds(my * k_my, k_my), :],
            preferred_element_type=jnp.float32,
        )

        for off in range(1, N_DEV):
            wcopies[off].wait()
            j = order[off]
            wb_ref[pl.ds(j * k_my, k_my), :] = (
                wv_ref[pl.ds(j * k_my, k_my), :].astype(jnp.bfloat16))

        for off in range(1, N_DEV):
            rdmas[off - 1].wait()
            j = order[off]
            out_ref[:, :] = out_ref[:, :] + jnp.dot(
                xg_ref[:, pl.ds(j * k_my, k_my)],
                wb_ref[pl.ds(j * k_my, k_my), :],
                preferred_element_type=jnp.float32,
            )

        acc = out_ref[:, :]
        c = 0.7978845608028654
        out_ref[:, :] = 0.5 * acc * (1.0 + jnp.tanh(c * (acc + 0.044715 * acc * acc * acc)))

    return pl.pallas_call(
        body,
        out_shape=jax.ShapeDtypeStruct((m_per, n), jnp.float32),
        in_specs=[
            pl.BlockSpec(memory_space=pltpu.VMEM),
            pl.BlockSpec(memory_space=pl.ANY),
        ],
        out_specs=pl.BlockSpec(memory_space=pltpu.VMEM),
        scratch_shapes=[
            pltpu.VMEM((m_global, k_my), jnp.bfloat16),
            pltpu.VMEM((m_per, k_global), jnp.bfloat16),
            pltpu.VMEM((k_global, n), jnp.float32),
            pltpu.VMEM((k_global, n), jnp.bfloat16),
            pltpu.SemaphoreType.DMA((N_DEV - 1,)),
            pltpu.SemaphoreType.DMA((N_DEV - 1,)),
            pltpu.SemaphoreType.DMA((N_DEV,)),
        ],
        compiler_params=pltpu.CompilerParams(collective_id=0),
    )(x, w_mat)
